# Initial kernel scaffold; baseline (speedup 1.0000x reference)
import jax
import jax.numpy as jnp
from jax.experimental import pallas as pl


def kernel(w0, w1, w2, w3, w4, w5, w6, w7):
    raise NotImplementedError("write your pallas kernel here")


# speedup vs baseline: 2.8028x; 2.8028x over previous
import jax
import jax.numpy as jnp
from jax import lax
from jax.experimental import pallas as pl
from jax.experimental.pallas import tpu as pltpu
from jax.experimental.pallas import tpu_sc as plsc

_LENS = (480000, 448000, 416000, 384000, 352000, 320000, 288000, 256000)
_MAXL = 480000
_NC, _NS = 2, 16
_NW = _NC * _NS
_A = 16

_BASE = tuple((l // _NW) & ~(_A - 1) for l in _LENS)
_DREM = tuple(l - _NW * b for l, b in zip(_LENS, _BASE))
_PBASE = tuple(((_MAXL - l) // _NW) & ~(_A - 1) for l in _LENS)
_PREM = tuple((_MAXL - l) - _NW * p for l, p in zip(_LENS, _PBASE))

_OFFS = tuple(sum(_BASE[:i]) for i in range(8))
_ROFFS = {}
_cur = sum(_BASE)
for _i in range(8):
    if _DREM[_i]:
        _ROFFS[_i] = _cur
        _cur += _DREM[_i]
_STAGE = _cur
_ZMAX = max(max(_PBASE), 256)


def _body(w0, w1, w2, w3, w4, w5, w6, w7, out, stage, zbuf, sem):
    ws = (w0, w1, w2, w3, w4, w5, w6, w7)
    wid = lax.axis_index("s") * _NC + lax.axis_index("c")

    def in_copy(i):
        return pltpu.make_async_copy(
            ws[i].at[pl.ds(wid * _BASE[i], _BASE[i])],
            stage.at[pl.ds(_OFFS[i], _BASE[i])],
            sem,
        )

    def in_rem_copy(i):
        return pltpu.make_async_copy(
            ws[i].at[pl.ds(_NW * _BASE[i], _DREM[i])],
            stage.at[pl.ds(_ROFFS[i], _DREM[i])],
            sem,
        )

    def out_copy(i):
        return pltpu.make_async_copy(
            stage.at[pl.ds(_OFFS[i], _BASE[i])],
            out.at[pl.ds(i * _MAXL + wid * _BASE[i], _BASE[i])],
            sem,
        )

    def out_rem_copy(i):
        return pltpu.make_async_copy(
            stage.at[pl.ds(_ROFFS[i], _DREM[i])],
            out.at[pl.ds(i * _MAXL + _NW * _BASE[i], _DREM[i])],
            sem,
        )

    def pad_copy(i):
        return pltpu.make_async_copy(
            zbuf.at[pl.ds(0, _PBASE[i])],
            out.at[pl.ds(i * _MAXL + _LENS[i] + wid * _PBASE[i], _PBASE[i])],
            sem,
        )

    def pad_rem_copy(i):
        return pltpu.make_async_copy(
            zbuf.at[pl.ds(0, _PREM[i])],
            out.at[pl.ds(i * _MAXL + _LENS[i] + _NW * _PBASE[i], _PREM[i])],
            sem,
        )

    for i in range(8):
        in_copy(i).start()
        if _DREM[i]:

            @pl.when(wid == i)
            def _(i=i):
                in_rem_copy(i).start()

    def _zstep(j, c):
        zbuf[pl.ds(j * _A, _A)] = jnp.zeros((_A,), jnp.float32)
        return c

    lax.fori_loop(0, _ZMAX // _A, _zstep, 0)

    for i in range(8):
        in_copy(i).wait()
        if _DREM[i]:

            @pl.when(wid == i)
            def _(i=i):
                in_rem_copy(i).wait()

    for i in range(8):
        out_copy(i).start()
        if _PBASE[i]:
            pad_copy(i).start()
        if _DREM[i]:

            @pl.when(wid == i)
            def _(i=i):
                out_rem_copy(i).start()

        if _PREM[i]:

            @pl.when(wid == i)
            def _(i=i):
                pad_rem_copy(i).start()

    for i in range(8):
        out_copy(i).wait()
        if _PBASE[i]:
            pad_copy(i).wait()
        if _DREM[i]:

            @pl.when(wid == i)
            def _(i=i):
                out_rem_copy(i).wait()

        if _PREM[i]:

            @pl.when(wid == i)
            def _(i=i):
                pad_rem_copy(i).wait()


@jax.jit
def _pad_stack(w0, w1, w2, w3, w4, w5, w6, w7):
    mesh = plsc.VectorSubcoreMesh(core_axis_name="c", subcore_axis_name="s")
    f = pl.kernel(
        _body,
        out_type=jax.ShapeDtypeStruct((8 * _MAXL,), jnp.float32),
        mesh=mesh,
        scratch_types=[
            pltpu.VMEM((_STAGE,), jnp.float32),
            pltpu.VMEM((_ZMAX,), jnp.float32),
            pltpu.SemaphoreType.DMA,
        ],
    )
    return f(w0, w1, w2, w3, w4, w5, w6, w7)


def kernel(w0, w1, w2, w3, w4, w5, w6, w7):
    batched = _pad_stack(w0, w1, w2, w3, w4, w5, w6, w7).reshape(8, _MAXL)
    wave_lengths = jnp.array(_LENS, dtype=jnp.int32)
    return (batched, wave_lengths)


# speedup vs baseline: 2.8198x; 1.0060x over previous
import jax
import jax.numpy as jnp
from jax import lax
from jax.experimental import pallas as pl
from jax.experimental.pallas import tpu as pltpu
from jax.experimental.pallas import tpu_sc as plsc

_LENS = (480000, 448000, 416000, 384000, 352000, 320000, 288000, 256000)
_MAXL = 480000
_NC, _NS = 2, 16
_NW = _NC * _NS
_A = 16

_BASE = tuple((l // _NW) & ~(_A - 1) for l in _LENS)
_DREM = tuple(l - _NW * b for l, b in zip(_LENS, _BASE))
_PBASE = tuple(((_MAXL - l) // _NW) & ~(_A - 1) for l in _LENS)
_PREM = tuple((_MAXL - l) - _NW * p for l, p in zip(_LENS, _PBASE))

_OFFS = tuple(sum(_BASE[:i]) for i in range(8))
_ROFFS = {}
_cur = sum(_BASE)
for _i in range(8):
    if _DREM[_i]:
        _ROFFS[_i] = _cur
        _cur += _DREM[_i]
_STAGE = _cur
_ZMAX = max(max(_PBASE), 256)


def _body(w0, w1, w2, w3, w4, w5, w6, w7, out, stage, zbuf, *sems):
    ws = (w0, w1, w2, w3, w4, w5, w6, w7)
    in_sems, osem = sems[:8], sems[8]
    wid = lax.axis_index("s") * _NC + lax.axis_index("c")

    def in_copy(i):
        return pltpu.make_async_copy(
            ws[i].at[pl.ds(wid * _BASE[i], _BASE[i])],
            stage.at[pl.ds(_OFFS[i], _BASE[i])],
            in_sems[i],
        )

    def in_rem_copy(i):
        return pltpu.make_async_copy(
            ws[i].at[pl.ds(_NW * _BASE[i], _DREM[i])],
            stage.at[pl.ds(_ROFFS[i], _DREM[i])],
            in_sems[i],
        )

    def out_copy(i):
        return pltpu.make_async_copy(
            stage.at[pl.ds(_OFFS[i], _BASE[i])],
            out.at[pl.ds(i * _MAXL + wid * _BASE[i], _BASE[i])],
            osem,
        )

    def out_rem_copy(i):
        return pltpu.make_async_copy(
            stage.at[pl.ds(_ROFFS[i], _DREM[i])],
            out.at[pl.ds(i * _MAXL + _NW * _BASE[i], _DREM[i])],
            osem,
        )

    def pad_copy(i):
        return pltpu.make_async_copy(
            zbuf.at[pl.ds(0, _PBASE[i])],
            out.at[pl.ds(i * _MAXL + _LENS[i] + wid * _PBASE[i], _PBASE[i])],
            osem,
        )

    def pad_rem_copy(i):
        return pltpu.make_async_copy(
            zbuf.at[pl.ds(0, _PREM[i])],
            out.at[pl.ds(i * _MAXL + _LENS[i] + _NW * _PBASE[i], _PREM[i])],
            osem,
        )

    for i in range(8):
        in_copy(i).start()
        if _DREM[i]:

            @pl.when(wid == i)
            def _(i=i):
                in_rem_copy(i).start()

    def _zstep(j, c):
        zbuf[pl.ds(j * _A, _A)] = jnp.zeros((_A,), jnp.float32)
        return c

    lax.fori_loop(0, _ZMAX // _A, _zstep, 0)

    for i in range(8):
        if _PBASE[i]:
            pad_copy(i).start()
        if _PREM[i]:

            @pl.when(wid == i)
            def _(i=i):
                pad_rem_copy(i).start()

    for i in range(8):
        in_copy(i).wait()
        if _DREM[i]:

            @pl.when(wid == i)
            def _(i=i):
                in_rem_copy(i).wait()

        out_copy(i).start()
        if _DREM[i]:

            @pl.when(wid == i)
            def _(i=i):
                out_rem_copy(i).start()

    for i in range(8):
        out_copy(i).wait()
        if _PBASE[i]:
            pad_copy(i).wait()
        if _DREM[i]:

            @pl.when(wid == i)
            def _(i=i):
                out_rem_copy(i).wait()

        if _PREM[i]:

            @pl.when(wid == i)
            def _(i=i):
                pad_rem_copy(i).wait()


@jax.jit
def _pad_stack(w0, w1, w2, w3, w4, w5, w6, w7):
    mesh = plsc.VectorSubcoreMesh(core_axis_name="c", subcore_axis_name="s")
    f = pl.kernel(
        _body,
        out_type=jax.ShapeDtypeStruct((8 * _MAXL,), jnp.float32),
        mesh=mesh,
        scratch_types=[
            pltpu.VMEM((_STAGE,), jnp.float32),
            pltpu.VMEM((_ZMAX,), jnp.float32),
        ] + [pltpu.SemaphoreType.DMA] * 9,
    )
    return f(w0, w1, w2, w3, w4, w5, w6, w7)


def kernel(w0, w1, w2, w3, w4, w5, w6, w7):
    batched = _pad_stack(w0, w1, w2, w3, w4, w5, w6, w7).reshape(8, _MAXL)
    wave_lengths = jnp.array(_LENS, dtype=jnp.int32)
    return (batched, wave_lengths)


# speedup vs baseline: 3.2161x; 1.1406x over previous
import jax
import jax.numpy as jnp
from jax import lax
from jax.experimental import pallas as pl
from jax.experimental.pallas import tpu as pltpu
from jax.experimental.pallas import tpu_sc as plsc

_LENS = (480000, 448000, 416000, 384000, 352000, 320000, 288000, 256000)
_MAXL = 480000
_NC, _NS = 2, 16
_NW = _NC * _NS
_W = 16000
_NT = _MAXL // _W
_TPC = 32000 // _W


def _body(w0, w1, w2, w3, w4, w5, w6, w7, zrow, out, buf, isem, osem):
    ws = (w0, w1, w2, w3, w4, w5, w6, w7)
    wid = lax.axis_index("s") * _NC + lax.axis_index("c")
    t = wid
    valid = t < _NT
    c0 = pl.multiple_of(t * _W, _W)

    def in_copy(r):
        return pltpu.make_async_copy(
            ws[r].at[pl.ds(c0, _W)], buf.at[r], isem
        )

    def zero_copy(r):
        return pltpu.make_async_copy(zrow, buf.at[r], isem)

    for r in range(8):
        data = t < (15 - r) * _TPC

        @pl.when(valid & data)
        def _(r=r):
            in_copy(r).start()

        @pl.when(valid & jnp.logical_not(data))
        def _(r=r):
            zero_copy(r).start()

    @pl.when(valid)
    def _():
        for r in range(8):
            in_copy(r).wait()

    out_copy = pltpu.make_async_copy(
        buf, out.at[:, pl.ds(c0, _W)], osem
    )

    @pl.when(valid)
    def _():
        out_copy.start()
        out_copy.wait()


@jax.jit
def _pad_stack(w0, w1, w2, w3, w4, w5, w6, w7):
    mesh = plsc.VectorSubcoreMesh(core_axis_name="c", subcore_axis_name="s")
    f = pl.kernel(
        _body,
        out_type=jax.ShapeDtypeStruct((8, _MAXL), jnp.float32),
        mesh=mesh,
        scratch_types=[
            pltpu.VMEM((8, _W), jnp.float32),
            pltpu.SemaphoreType.DMA,
            pltpu.SemaphoreType.DMA,
        ],
    )
    zrow = jnp.zeros((_W,), jnp.float32)
    return f(w0, w1, w2, w3, w4, w5, w6, w7, zrow)


def kernel(w0, w1, w2, w3, w4, w5, w6, w7):
    batched = _pad_stack(w0, w1, w2, w3, w4, w5, w6, w7)
    wave_lengths = jnp.array(_LENS, dtype=jnp.int32)
    return (batched, wave_lengths)
